# baseline (speedup 1.0000x reference)
import math

import functools

import jax
import jax.numpy as jnp
from jax import lax
from jax.experimental import pallas as pl
from jax.experimental.pallas import tpu as pltpu
from jax.experimental.pallas import tpu_sc as plsc

_C = 100000
_SMOOTHING = 0.1
_CONF = 1.0 - _SMOOTHING
_EPS = _SMOOTHING / (_C - 1)

_R = 8
_L = 128
_NT = -(-_C // _L)
_CPAD = _NT * _L
_HT = _NT // 2
_HPAD = _HT * _L
_TAIL_REM = _C - (_NT - 1) * _L
_NCH = 16
_SEC = 8192
_NSEC = -(-_C // _SEC)
_NEG_INF = float("-inf")

_B = 1024
_B_SC = 256
_ROW0 = _B - _B_SC
_NW = 32
_RPW = _B_SC // _NW
_CH = 8192
_NCHK = _C // _CH
_CALIGN = 128
_CTAIL = ((_C - _NCHK * _CH) // _CALIGN) * _CALIGN
_REM = _C - _NCHK * _CH - _CTAIL


def _tree(vals, op):
    while len(vals) > 1:
        nxt = [op(vals[i], vals[i + 1]) for i in range(0, len(vals) - 1, 2)]
        if len(vals) % 2:
            nxt.append(vals[-1])
        vals = nxt
    return vals[0]


def _tile(refs, t):
    if t < _HT:
        return refs[0][:, pl.ds(t * _L, _L)]
    return refs[1][:, pl.ds((t - _HT) * _L, _L)]


def _lane_iota():
    return jax.lax.broadcasted_iota(jnp.int32, (_R, _L), 1)


def _chains(n):
    per = -(-n // _NCH)
    for c in range(_NCH):
        lo = c * per
        hi = min(lo + per, n)
        if lo < hi:
            yield range(lo, hi)


def _masked(x, t, fill):
    if t == _NT - 1:
        return jnp.where(_lane_iota() < _TAIL_REM, x, fill)
    return x


def _sweep_max(pred_ref):
    accs = []
    for chain in _chains(_NT):
        acc = None
        for t in chain:
            x = _masked(_tile(pred_ref, t), t, _NEG_INF)
            acc = x if acc is None else jnp.maximum(acc, x)
        accs.append(acc)
    return _tree(accs, jnp.maximum)


def _sweep_stats(pred_ref, m):
    se_accs = []
    sp_accs = []
    for chain in _chains(_NT):
        se = None
        sp = None
        for t in chain:
            x = _tile(pred_ref, t)
            e = jnp.exp(_masked(x, t, _NEG_INF) - m)
            xs = _masked(x, t, 0.0)
            se = e if se is None else se + e
            sp = xs if sp is None else sp + xs
        se_accs.append(se)
        sp_accs.append(sp)
    return _tree(se_accs, jnp.add), _tree(sp_accs, jnp.add)


def _loss_kernel(tgt_s_ref, tgt_v_ref, pred_lo_ref, pred_hi_ref, out_ref, ts_ref):
    pred_ref = (pred_lo_ref, pred_hi_ref)
    rb = pl.program_id(0)

    ts_ref[...] = jnp.zeros((_R, _L), jnp.float32)

    tgtv = jnp.broadcast_to(tgt_v_ref[0, 0, :].reshape(_R, 1), (_R, _L))
    li = _lane_iota()
    for sec in range(_NSEC):
        lo = sec * _SEC
        hi = min(lo + _SEC, _C)
        hit = None
        for i in range(_R):
            t = tgt_s_ref[0, 0, i]
            h = jnp.logical_and(t >= lo, t < hi)
            hit = h if hit is None else jnp.logical_or(hit, h)

        @pl.when(hit)
        def _tsel(lo=lo, hi=hi):
            accs = []
            t0 = lo // _L
            t1 = -(-hi // _L)
            for g0 in range(t0, t1, 8):
                acc = None
                for t in range(g0, min(g0 + 8, t1)):
                    col = li + t * _L
                    v = jnp.where(col == tgtv, _tile(pred_ref, t), 0.0)
                    acc = v if acc is None else acc + v
                accs.append(acc)
            ts_ref[...] = ts_ref[...] + _tree(accs, jnp.add)

    m = _sweep_max(pred_ref)
    se, sp = _sweep_stats(pred_ref, m)

    mx = jnp.max(m, axis=1, keepdims=True)
    sx = jnp.sum(se * jnp.exp(m - mx), axis=1, keepdims=True)
    spx = jnp.sum(sp, axis=1, keepdims=True)
    tsx = jnp.sum(ts_ref[...], axis=1, keepdims=True)
    lse = mx + jnp.log(sx)
    rowsum_logp = spx - _C * lse
    logp_t = tsx - lse
    contrib = -(_EPS * rowsum_logp + (_CONF - _EPS) * logp_t)
    val = jnp.sum(contrib)

    @pl.when(rb == 0)
    def _():
        out_ref[0, 0] = val

    @pl.when(rb > 0)
    def _():
        out_ref[0, 0] = out_ref[0, 0] + val


def _sc_mesh():
    return plsc.VectorSubcoreMesh(core_axis_name="c", subcore_axis_name="s")


@functools.partial(
    pl.kernel,
    out_type=[
        jax.ShapeDtypeStruct((_B_SC * 16,), jnp.float32),
        jax.ShapeDtypeStruct((_B_SC * 16,), jnp.float32),
        jax.ShapeDtypeStruct((_B_SC * 16,), jnp.float32),
        jax.ShapeDtypeStruct((_B_SC * 16,), jnp.float32),
    ],
    mesh=_sc_mesh(),
    scratch_types=[
        pltpu.VMEM((_CH,), jnp.float32),
        pltpu.VMEM((16,), jnp.int32),
        pltpu.VMEM((_RPW * 16,), jnp.float32),
        pltpu.VMEM((_RPW * 16,), jnp.float32),
        pltpu.VMEM((_RPW * 16,), jnp.float32),
        pltpu.VMEM((_RPW * 16,), jnp.float32),
    ],
)
def _sc_stats(pred_hbm, tail_hbm, tgt_hbm, m_out, s_out, sp_out, ts_out,
              buf, tbuf, mrow, srow, sprow, tsrow):
    wid = lax.axis_index("s") * 2 + lax.axis_index("c")
    base_row = _ROW0 + wid * _RPW
    pltpu.sync_copy(tgt_hbm.at[pl.ds(base_row, _RPW)],
                    tbuf.at[pl.ds(0, _RPW)])
    tvec = tbuf[...]
    lanes = jnp.arange(16, dtype=jnp.int32)

    for i in range(_RPW):
        row = base_row + i
        t = tvec[i]
        m = jnp.full((16,), _NEG_INF, jnp.float32)
        s = jnp.zeros((16,), jnp.float32)
        sp = jnp.zeros((16,), jnp.float32)
        tsel_v = jnp.zeros((16,), jnp.float32)
        for c in range(_NCHK + 1):
            ln = _CH if c < _NCHK else _CTAIL
            nv = ln // 16
            pltpu.sync_copy(pred_hbm.at[row, pl.ds(c * _CH, ln)],
                            buf.at[pl.ds(0, ln)])

            def mx_body(j, acc):
                return jnp.maximum(acc, buf[pl.ds(j * 16, 16)])

            cmax = lax.fori_loop(
                0, nv, mx_body, jnp.full((16,), _NEG_INF, jnp.float32))
            m_new = jnp.maximum(m, cmax)

            def st_body(j, io):
                se_a, sp_a = io
                x = buf[pl.ds(j * 16, 16)]
                return (se_a + jnp.exp(x - m_new), sp_a + x)

            se_c, sp_c = lax.fori_loop(
                0, nv, st_body,
                (jnp.zeros((16,), jnp.float32),
                 jnp.zeros((16,), jnp.float32)))
            s = s * jnp.exp(m - m_new) + se_c
            m = m_new
            sp = sp + sp_c

            off = t - c * _CH
            off_al = jnp.clip((off // 16) * 16, 0, ln - 16)
            v16 = buf[pl.ds(off_al, 16)]
            d = lanes - (off - off_al)
            tsel_v = tsel_v + jnp.where(d == 0, v16, 0.0)

        pltpu.sync_copy(tail_hbm.at[row], buf.at[pl.ds(0, _CALIGN)])
        x1 = buf[pl.ds(_CALIGN - 32, 16)]
        x2 = buf[pl.ds(_CALIGN - 16, 16)]
        m_new = jnp.maximum(m, jnp.maximum(x1, x2))
        se_c = jnp.exp(x1 - m_new) + jnp.exp(x2 - m_new)
        s = s * jnp.exp(m - m_new) + se_c
        m = m_new
        sp = sp + x1 + x2
        off = t - (_C - 32)
        tsel_v = tsel_v + jnp.where(lanes - off == 0, x1, 0.0)
        tsel_v = tsel_v + jnp.where(lanes - (off - 16) == 0, x2, 0.0)

        mrow[pl.ds(i * 16, 16)] = m
        srow[pl.ds(i * 16, 16)] = s
        sprow[pl.ds(i * 16, 16)] = sp
        tsrow[pl.ds(i * 16, 16)] = tsel_v

    out_lo = wid * _RPW * 16
    pltpu.sync_copy(mrow, m_out.at[pl.ds(out_lo, _RPW * 16)])
    pltpu.sync_copy(srow, s_out.at[pl.ds(out_lo, _RPW * 16)])
    pltpu.sync_copy(sprow, sp_out.at[pl.ds(out_lo, _RPW * 16)])
    pltpu.sync_copy(tsrow, ts_out.at[pl.ds(out_lo, _RPW * 16)])


@jax.jit
def kernel(pred, target):
    tgt32 = target.astype(jnp.int32)
    nb = _ROW0 // _R
    tgt3 = tgt32[:_ROW0].reshape(nb, 1, _R)

    acc = pl.pallas_call(
        _loss_kernel,
        grid=(nb,),
        in_specs=[
            pl.BlockSpec((1, 1, _R), lambda rb: (rb, 0, 0),
                         memory_space=pltpu.SMEM),
            pl.BlockSpec((1, 1, _R), lambda rb: (rb, 0, 0)),
            pl.BlockSpec((_R, _HPAD), lambda rb: (rb, 0)),
            pl.BlockSpec((_R, _HPAD), lambda rb: (rb, 1)),
        ],
        out_specs=pl.BlockSpec(
            (1, 1), lambda rb: (0, 0), memory_space=pltpu.SMEM),
        out_shape=jax.ShapeDtypeStruct((1, 1), jnp.float32),
        scratch_shapes=[
            pltpu.VMEM((_R, _L), jnp.float32),
        ],
    )(tgt3, tgt3, pred, pred)

    pred_tail = lax.slice(pred, (0, _C - _CALIGN), (_B, _C))
    m_sc, s_sc, sp_sc, ts_sc = _sc_stats(pred, pred_tail, tgt32)
    m_sc = m_sc.reshape(_B_SC, 16)
    s_sc = s_sc.reshape(_B_SC, 16)
    sp_sc = sp_sc.reshape(_B_SC, 16)
    ts_sc = ts_sc.reshape(_B_SC, 16)
    mx_sc = jnp.max(m_sc, axis=1)
    s_row = jnp.sum(s_sc * jnp.exp(m_sc - mx_sc[:, None]), axis=1)
    sp_row = jnp.sum(sp_sc, axis=1)
    ts_row = jnp.sum(ts_sc, axis=1)
    lse_sc = mx_sc + jnp.log(s_row)
    contrib_sc = -(_EPS * (sp_row - _C * lse_sc)
                   + (_CONF - _EPS) * (ts_row - lse_sc))

    k0 = (_C - 1) * _EPS * math.log(_EPS) + _CONF * math.log(_CONF)
    return (acc[0, 0] + jnp.sum(contrib_sc) + _B * k0) / (_B * _C)


# speedup vs baseline: 1.0002x; 1.0002x over previous
import math

import functools

import jax
import jax.numpy as jnp
from jax import lax
from jax.experimental import pallas as pl
from jax.experimental.pallas import tpu as pltpu
from jax.experimental.pallas import tpu_sc as plsc

_C = 100000
_SMOOTHING = 0.1
_CONF = 1.0 - _SMOOTHING
_EPS = _SMOOTHING / (_C - 1)

_R = 8
_L = 128
_NT = -(-_C // _L)
_CPAD = _NT * _L
_HT = _NT // 2
_HPAD = _HT * _L
_TAIL_REM = _C - (_NT - 1) * _L
_NCH = 16
_SEC = 8192
_NSEC = -(-_C // _SEC)
_NEG_INF = float("-inf")

_B = 1024
_B_SC = 256
_ROW0 = _B - _B_SC
_NW = 32
_RPW = _B_SC // _NW
_CH = 8192
_NCHK = _C // _CH
_CALIGN = 128
_CTAIL = ((_C - _NCHK * _CH) // _CALIGN) * _CALIGN
_REM = _C - _NCHK * _CH - _CTAIL


def _tree(vals, op):
    while len(vals) > 1:
        nxt = [op(vals[i], vals[i + 1]) for i in range(0, len(vals) - 1, 2)]
        if len(vals) % 2:
            nxt.append(vals[-1])
        vals = nxt
    return vals[0]


def _tile(refs, t):
    if t < _HT:
        return refs[0][:, pl.ds(t * _L, _L)]
    return refs[1][:, pl.ds((t - _HT) * _L, _L)]


def _lane_iota():
    return jax.lax.broadcasted_iota(jnp.int32, (_R, _L), 1)


def _chains(n):
    per = -(-n // _NCH)
    for c in range(_NCH):
        lo = c * per
        hi = min(lo + per, n)
        if lo < hi:
            yield range(lo, hi)


def _masked(x, t, fill):
    if t == _NT - 1:
        return jnp.where(_lane_iota() < _TAIL_REM, x, fill)
    return x


def _sweep_max(pred_ref):
    accs = []
    for chain in _chains(_NT):
        acc = None
        for t in chain:
            x = _masked(_tile(pred_ref, t), t, _NEG_INF)
            acc = x if acc is None else jnp.maximum(acc, x)
        accs.append(acc)
    return _tree(accs, jnp.maximum)


def _sweep_stats(pred_ref, m):
    se_accs = []
    sp_accs = []
    for chain in _chains(_NT):
        se = None
        sp = None
        for t in chain:
            x = _tile(pred_ref, t)
            e = jnp.exp(_masked(x, t, _NEG_INF) - m)
            xs = _masked(x, t, 0.0)
            se = e if se is None else se + e
            sp = xs if sp is None else sp + xs
        se_accs.append(se)
        sp_accs.append(sp)
    return _tree(se_accs, jnp.add), _tree(sp_accs, jnp.add)


def _loss_kernel(tgt_s_ref, tgt_v_ref, pred_lo_ref, pred_hi_ref, out_ref, ts_ref):
    pred_ref = (pred_lo_ref, pred_hi_ref)
    rb = pl.program_id(0)

    ts_ref[...] = jnp.zeros((_R, _L), jnp.float32)

    tgtv = jnp.broadcast_to(tgt_v_ref[0, 0, :].reshape(_R, 1), (_R, _L))
    li = _lane_iota()
    for sec in range(_NSEC):
        lo = sec * _SEC
        hi = min(lo + _SEC, _C)
        hit = None
        for i in range(_R):
            t = tgt_s_ref[0, 0, i]
            h = jnp.logical_and(t >= lo, t < hi)
            hit = h if hit is None else jnp.logical_or(hit, h)

        @pl.when(hit)
        def _tsel(lo=lo, hi=hi):
            accs = []
            t0 = lo // _L
            t1 = -(-hi // _L)
            for g0 in range(t0, t1, 8):
                acc = None
                for t in range(g0, min(g0 + 8, t1)):
                    col = li + t * _L
                    v = jnp.where(col == tgtv, _tile(pred_ref, t), 0.0)
                    acc = v if acc is None else acc + v
                accs.append(acc)
            ts_ref[...] = ts_ref[...] + _tree(accs, jnp.add)

    m = _sweep_max(pred_ref)
    se, sp = _sweep_stats(pred_ref, m)

    mx = jnp.max(m, axis=1, keepdims=True)
    sx = jnp.sum(se * jnp.exp(m - mx), axis=1, keepdims=True)
    spx = jnp.sum(sp, axis=1, keepdims=True)
    tsx = jnp.sum(ts_ref[...], axis=1, keepdims=True)
    lse = mx + jnp.log(sx)
    rowsum_logp = spx - _C * lse
    logp_t = tsx - lse
    contrib = -(_EPS * rowsum_logp + (_CONF - _EPS) * logp_t)
    val = jnp.sum(contrib)

    @pl.when(rb == 0)
    def _():
        out_ref[0, 0] = val

    @pl.when(rb > 0)
    def _():
        out_ref[0, 0] = out_ref[0, 0] + val


def _sc_mesh():
    return plsc.VectorSubcoreMesh(core_axis_name="c", subcore_axis_name="s")


@functools.partial(
    pl.kernel,
    out_type=[
        jax.ShapeDtypeStruct((_B_SC * 16,), jnp.float32),
        jax.ShapeDtypeStruct((_B_SC * 16,), jnp.float32),
        jax.ShapeDtypeStruct((_B_SC * 16,), jnp.float32),
        jax.ShapeDtypeStruct((_B_SC * 16,), jnp.float32),
    ],
    mesh=_sc_mesh(),
    scratch_types=[
        pltpu.VMEM((_CH,), jnp.float32),
        pltpu.VMEM((16,), jnp.int32),
        pltpu.VMEM((_RPW * 16,), jnp.float32),
        pltpu.VMEM((_RPW * 16,), jnp.float32),
        pltpu.VMEM((_RPW * 16,), jnp.float32),
        pltpu.VMEM((_RPW * 16,), jnp.float32),
    ],
)
def _sc_stats(pred_hbm, tail_hbm, tgt_hbm, m_out, s_out, sp_out, ts_out,
              buf, tbuf, mrow, srow, sprow, tsrow):
    wid = lax.axis_index("s") * 2 + lax.axis_index("c")
    base_row = _ROW0 + wid * _RPW
    pltpu.sync_copy(tgt_hbm.at[pl.ds(base_row, _RPW)],
                    tbuf.at[pl.ds(0, _RPW)])
    tvec = tbuf[...]
    lanes = jnp.arange(16, dtype=jnp.int32)

    for i in range(_RPW):
        row = base_row + i
        t = tvec[i]
        m = jnp.full((16,), _NEG_INF, jnp.float32)
        s = jnp.zeros((16,), jnp.float32)
        sp = jnp.zeros((16,), jnp.float32)
        tsel_v = jnp.zeros((16,), jnp.float32)
        for c in range(_NCHK + 1):
            ln = _CH if c < _NCHK else _CTAIL
            nv = ln // 16
            pltpu.sync_copy(pred_hbm.at[row, pl.ds(c * _CH, ln)],
                            buf.at[pl.ds(0, ln)])

            def mx_body(j, acc):
                return jnp.maximum(acc, buf[pl.ds(j * 16, 16)])

            cmax = lax.fori_loop(
                0, nv, mx_body, jnp.full((16,), _NEG_INF, jnp.float32))
            m_new = jnp.maximum(m, cmax)

            def st_body(j, io):
                se_a, sp_a = io
                x = buf[pl.ds(j * 16, 16)]
                return (se_a + jnp.exp(x - m_new), sp_a + x)

            se_c, sp_c = lax.fori_loop(
                0, nv, st_body,
                (jnp.zeros((16,), jnp.float32),
                 jnp.zeros((16,), jnp.float32)))
            s = s * jnp.exp(m - m_new) + se_c
            m = m_new
            sp = sp + sp_c

            off = t - c * _CH
            off_al = jnp.clip((off // 16) * 16, 0, ln - 16)
            v16 = buf[pl.ds(off_al, 16)]
            d = lanes - (off - off_al)
            tsel_v = tsel_v + jnp.where(d == 0, v16, 0.0)

        pltpu.sync_copy(tail_hbm.at[row], buf.at[pl.ds(0, _CALIGN)])
        x1 = buf[pl.ds(_CALIGN - 32, 16)]
        x2 = buf[pl.ds(_CALIGN - 16, 16)]
        m_new = jnp.maximum(m, jnp.maximum(x1, x2))
        se_c = jnp.exp(x1 - m_new) + jnp.exp(x2 - m_new)
        s = s * jnp.exp(m - m_new) + se_c
        m = m_new
        sp = sp + x1 + x2
        off = t - (_C - 32)
        tsel_v = tsel_v + jnp.where(lanes - off == 0, x1, 0.0)
        tsel_v = tsel_v + jnp.where(lanes - (off - 16) == 0, x2, 0.0)

        mrow[pl.ds(i * 16, 16)] = m
        srow[pl.ds(i * 16, 16)] = s
        sprow[pl.ds(i * 16, 16)] = sp
        tsrow[pl.ds(i * 16, 16)] = tsel_v

    out_lo = wid * _RPW * 16
    pltpu.sync_copy(mrow, m_out.at[pl.ds(out_lo, _RPW * 16)])
    pltpu.sync_copy(srow, s_out.at[pl.ds(out_lo, _RPW * 16)])
    pltpu.sync_copy(sprow, sp_out.at[pl.ds(out_lo, _RPW * 16)])
    pltpu.sync_copy(tsrow, ts_out.at[pl.ds(out_lo, _RPW * 16)])


@jax.jit
def kernel(pred, target):
    tgt32 = target.astype(jnp.int32)
    nb = _ROW0 // _R
    tgt3 = tgt32[:_ROW0].reshape(nb, 1, _R)

    pred_tail = lax.slice(pred, (0, _C - _CALIGN), (_B, _C))
    m_sc, s_sc, sp_sc, ts_sc = _sc_stats(pred, pred_tail, tgt32)

    acc = pl.pallas_call(
        _loss_kernel,
        grid=(nb,),
        in_specs=[
            pl.BlockSpec((1, 1, _R), lambda rb: (rb, 0, 0),
                         memory_space=pltpu.SMEM),
            pl.BlockSpec((1, 1, _R), lambda rb: (rb, 0, 0)),
            pl.BlockSpec((_R, _HPAD), lambda rb: (rb, 0)),
            pl.BlockSpec((_R, _HPAD), lambda rb: (rb, 1)),
        ],
        out_specs=pl.BlockSpec(
            (1, 1), lambda rb: (0, 0), memory_space=pltpu.SMEM),
        out_shape=jax.ShapeDtypeStruct((1, 1), jnp.float32),
        scratch_shapes=[
            pltpu.VMEM((_R, _L), jnp.float32),
        ],
    )(tgt3, tgt3, pred, pred)

    m_sc = m_sc.reshape(_B_SC, 16)
    s_sc = s_sc.reshape(_B_SC, 16)
    sp_sc = sp_sc.reshape(_B_SC, 16)
    ts_sc = ts_sc.reshape(_B_SC, 16)
    mx_sc = jnp.max(m_sc, axis=1)
    s_row = jnp.sum(s_sc * jnp.exp(m_sc - mx_sc[:, None]), axis=1)
    sp_row = jnp.sum(sp_sc, axis=1)
    ts_row = jnp.sum(ts_sc, axis=1)
    lse_sc = mx_sc + jnp.log(s_row)
    contrib_sc = -(_EPS * (sp_row - _C * lse_sc)
                   + (_CONF - _EPS) * (ts_row - lse_sc))

    k0 = (_C - 1) * _EPS * math.log(_EPS) + _CONF * math.log(_CONF)
    return (acc[0, 0] + jnp.sum(contrib_sc) + _B * k0) / (_B * _C)


# speedup vs baseline: 1.6677x; 1.6674x over previous
import math

import jax
import jax.numpy as jnp
from jax.experimental import pallas as pl
from jax.experimental.pallas import tpu as pltpu

_C = 100000
_SMOOTHING = 0.1
_CONF = 1.0 - _SMOOTHING
_EPS = _SMOOTHING / (_C - 1)

_R = 16
_L = 128
_NT = -(-_C // _L)
_CPAD = _NT * _L
_TAIL_REM = _C - (_NT - 1) * _L
_NCH = 16
_SEC = 8192
_NSEC = -(-_C // _SEC)
_NEG_INF = float("-inf")


def _tree(vals, op):
    while len(vals) > 1:
        nxt = [op(vals[i], vals[i + 1]) for i in range(0, len(vals) - 1, 2)]
        if len(vals) % 2:
            nxt.append(vals[-1])
        vals = nxt
    return vals[0]


def _tile(pred_ref, t):
    return pred_ref[:, pl.ds(t * _L, _L)]


def _lane_iota():
    return jax.lax.broadcasted_iota(jnp.int32, (_R, _L), 1)


def _chains(n):
    per = -(-n // _NCH)
    for c in range(_NCH):
        lo = c * per
        hi = min(lo + per, n)
        if lo < hi:
            yield range(lo, hi)


def _masked(x, t, fill):
    if t == _NT - 1:
        return jnp.where(_lane_iota() < _TAIL_REM, x, fill)
    return x


def _sweep_max(pred_ref):
    accs = []
    for chain in _chains(_NT):
        acc = None
        for t in chain:
            x = _masked(_tile(pred_ref, t), t, _NEG_INF)
            acc = x if acc is None else jnp.maximum(acc, x)
        accs.append(acc)
    return _tree(accs, jnp.maximum)


def _sweep_stats(pred_ref, m):
    se_accs = []
    sp_accs = []
    for chain in _chains(_NT):
        se = None
        sp = None
        for t in chain:
            x = _tile(pred_ref, t)
            e = jnp.exp(_masked(x, t, _NEG_INF) - m)
            xs = _masked(x, t, 0.0)
            se = e if se is None else se + e
            sp = xs if sp is None else sp + xs
        se_accs.append(se)
        sp_accs.append(sp)
    return _tree(se_accs, jnp.add), _tree(sp_accs, jnp.add)


def _loss_kernel(tgt_s_ref, tgt_v_ref, pred_ref, out_ref, ts_ref):
    rb = pl.program_id(0)

    ts_ref[...] = jnp.zeros((_R, _L), jnp.float32)

    tgtv = jnp.broadcast_to(tgt_v_ref[0, 0, :].reshape(_R, 1), (_R, _L))
    li = _lane_iota()
    for sec in range(_NSEC):
        lo = sec * _SEC
        hi = min(lo + _SEC, _C)
        hit = None
        for i in range(_R):
            t = tgt_s_ref[0, 0, i]
            h = jnp.logical_and(t >= lo, t < hi)
            hit = h if hit is None else jnp.logical_or(hit, h)

        @pl.when(hit)
        def _tsel(lo=lo, hi=hi):
            accs = []
            t0 = lo // _L
            t1 = -(-hi // _L)
            for g0 in range(t0, t1, 8):
                acc = None
                for t in range(g0, min(g0 + 8, t1)):
                    col = li + t * _L
                    v = jnp.where(col == tgtv, _tile(pred_ref, t), 0.0)
                    acc = v if acc is None else acc + v
                accs.append(acc)
            ts_ref[...] = ts_ref[...] + _tree(accs, jnp.add)

    m = _sweep_max(pred_ref)
    se, sp = _sweep_stats(pred_ref, m)

    mx = jnp.max(m, axis=1, keepdims=True)
    sx = jnp.sum(se * jnp.exp(m - mx), axis=1, keepdims=True)
    spx = jnp.sum(sp, axis=1, keepdims=True)
    tsx = jnp.sum(ts_ref[...], axis=1, keepdims=True)
    lse = mx + jnp.log(sx)
    rowsum_logp = spx - _C * lse
    logp_t = tsx - lse
    contrib = -(_EPS * rowsum_logp + (_CONF - _EPS) * logp_t)
    val = jnp.sum(contrib)

    @pl.when(rb == 0)
    def _():
        out_ref[0, 0] = val

    @pl.when(rb > 0)
    def _():
        out_ref[0, 0] = out_ref[0, 0] + val


@jax.jit
def kernel(pred, target):
    B = pred.shape[0]
    nb = B // _R
    tgt3 = target.astype(jnp.int32).reshape(nb, 1, _R)

    acc = pl.pallas_call(
        _loss_kernel,
        grid=(nb,),
        in_specs=[
            pl.BlockSpec((1, 1, _R), lambda rb: (rb, 0, 0),
                         memory_space=pltpu.SMEM),
            pl.BlockSpec((1, 1, _R), lambda rb: (rb, 0, 0)),
            pl.BlockSpec((_R, _CPAD), lambda rb: (rb, 0)),
        ],
        out_specs=pl.BlockSpec(
            (1, 1), lambda rb: (0, 0), memory_space=pltpu.SMEM),
        out_shape=jax.ShapeDtypeStruct((1, 1), jnp.float32),
        scratch_shapes=[
            pltpu.VMEM((_R, _L), jnp.float32),
        ],
    )(tgt3, tgt3, pred)

    k0 = (_C - 1) * _EPS * math.log(_EPS) + _CONF * math.log(_CONF)
    return (acc[0, 0] + B * k0) / (B * _C)


# speedup vs baseline: 1.7099x; 1.0253x over previous
import math

import jax
import jax.numpy as jnp
from jax.experimental import pallas as pl
from jax.experimental.pallas import tpu as pltpu

_C = 100000
_SMOOTHING = 0.1
_CONF = 1.0 - _SMOOTHING
_EPS = _SMOOTHING / (_C - 1)

_R = 32
_L = 128
_NT = -(-_C // _L)
_CPAD = _NT * _L
_TAIL_REM = _C - (_NT - 1) * _L
_NCH = 16
_SEC = 8192
_NSEC = -(-_C // _SEC)
_NEG_INF = float("-inf")


def _tree(vals, op):
    while len(vals) > 1:
        nxt = [op(vals[i], vals[i + 1]) for i in range(0, len(vals) - 1, 2)]
        if len(vals) % 2:
            nxt.append(vals[-1])
        vals = nxt
    return vals[0]


def _tile(pred_ref, t):
    return pred_ref[:, pl.ds(t * _L, _L)]


def _lane_iota():
    return jax.lax.broadcasted_iota(jnp.int32, (_R, _L), 1)


def _chains(n):
    per = -(-n // _NCH)
    for c in range(_NCH):
        lo = c * per
        hi = min(lo + per, n)
        if lo < hi:
            yield range(lo, hi)


def _masked(x, t, fill):
    if t == _NT - 1:
        return jnp.where(_lane_iota() < _TAIL_REM, x, fill)
    return x


def _sweep_max(pred_ref):
    accs = []
    for chain in _chains(_NT):
        acc = None
        for t in chain:
            x = _masked(_tile(pred_ref, t), t, _NEG_INF)
            acc = x if acc is None else jnp.maximum(acc, x)
        accs.append(acc)
    return _tree(accs, jnp.maximum)


def _sweep_stats(pred_ref, m):
    se_accs = []
    sp_accs = []
    for chain in _chains(_NT):
        se = None
        sp = None
        for t in chain:
            x = _tile(pred_ref, t)
            e = jnp.exp(_masked(x, t, _NEG_INF) - m)
            xs = _masked(x, t, 0.0)
            se = e if se is None else se + e
            sp = xs if sp is None else sp + xs
        se_accs.append(se)
        sp_accs.append(sp)
    return _tree(se_accs, jnp.add), _tree(sp_accs, jnp.add)


def _loss_kernel(tgt_s_ref, tgt_v_ref, pred_ref, out_ref, ts_ref):
    rb = pl.program_id(0)

    ts_ref[...] = jnp.zeros((_R, _L), jnp.float32)

    tgtv = jnp.broadcast_to(tgt_v_ref[0, 0, :].reshape(_R, 1), (_R, _L))
    li = _lane_iota()
    for sec in range(_NSEC):
        lo = sec * _SEC
        hi = min(lo + _SEC, _C)
        hit = None
        for i in range(_R):
            t = tgt_s_ref[0, 0, i]
            h = jnp.logical_and(t >= lo, t < hi)
            hit = h if hit is None else jnp.logical_or(hit, h)

        @pl.when(hit)
        def _tsel(lo=lo, hi=hi):
            accs = []
            t0 = lo // _L
            t1 = -(-hi // _L)
            for g0 in range(t0, t1, 8):
                acc = None
                for t in range(g0, min(g0 + 8, t1)):
                    col = li + t * _L
                    v = jnp.where(col == tgtv, _tile(pred_ref, t), 0.0)
                    acc = v if acc is None else acc + v
                accs.append(acc)
            ts_ref[...] = ts_ref[...] + _tree(accs, jnp.add)

    m = _sweep_max(pred_ref)
    se, sp = _sweep_stats(pred_ref, m)

    mx = jnp.max(m, axis=1, keepdims=True)
    sx = jnp.sum(se * jnp.exp(m - mx), axis=1, keepdims=True)
    spx = jnp.sum(sp, axis=1, keepdims=True)
    tsx = jnp.sum(ts_ref[...], axis=1, keepdims=True)
    lse = mx + jnp.log(sx)
    rowsum_logp = spx - _C * lse
    logp_t = tsx - lse
    contrib = -(_EPS * rowsum_logp + (_CONF - _EPS) * logp_t)
    val = jnp.sum(contrib)

    @pl.when(rb == 0)
    def _():
        out_ref[0, 0] = val

    @pl.when(rb > 0)
    def _():
        out_ref[0, 0] = out_ref[0, 0] + val


@jax.jit
def kernel(pred, target):
    B = pred.shape[0]
    nb = B // _R
    tgt3 = target.astype(jnp.int32).reshape(nb, 1, _R)

    acc = pl.pallas_call(
        _loss_kernel,
        grid=(nb,),
        in_specs=[
            pl.BlockSpec((1, 1, _R), lambda rb: (rb, 0, 0),
                         memory_space=pltpu.SMEM),
            pl.BlockSpec((1, 1, _R), lambda rb: (rb, 0, 0)),
            pl.BlockSpec((_R, _CPAD), lambda rb: (rb, 0)),
        ],
        out_specs=pl.BlockSpec(
            (1, 1), lambda rb: (0, 0), memory_space=pltpu.SMEM),
        out_shape=jax.ShapeDtypeStruct((1, 1), jnp.float32),
        scratch_shapes=[
            pltpu.VMEM((_R, _L), jnp.float32),
        ],
    )(tgt3, tgt3, pred)

    k0 = (_C - 1) * _EPS * math.log(_EPS) + _CONF * math.log(_CONF)
    return (acc[0, 0] + B * k0) / (B * _C)
